# baseline (device time: 231344 ns/iter reference)
import os

import jax
import jax.numpy as jnp
from jax import lax
from jax.experimental import pallas as pl
from jax.experimental.pallas import tpu as pltpu

N_DEV = 8
M_PER = 512
N_PER = 1024
K = 4096

_KVAR = os.environ.get("KVAR", "full")
_DO_COMPUTE = _KVAR != "comm"
_DO_COMM = _KVAR != "compute"


def kernel(x, w_mat, scale_x, scale_w):
    x8 = x.astype(jnp.float8_e5m2)
    w8 = w_mat.astype(jnp.float8_e5m2)
    scale = (scale_x[0] * scale_w[0]).reshape(1, 1).astype(jnp.float32)

    def body(x_ref, w_ref, s_ref, out_ref, sendbuf, send_sems, recv_sems):
        my = lax.axis_index("i")
        s = s_ref[0, 0]

        def block(col):
            acc = jnp.dot(
                x_ref[...],
                w_ref[:, pl.ds(col * N_PER, N_PER)],
                preferred_element_type=jnp.float32,
            )
            y = acc * s
            return y * jax.nn.sigmoid(y)

        sends = []
        for t in range(1, N_DEV):
            j = (my + t) % N_DEV
            slot = (t - 1) % 6
            if t == 7 and _DO_COMM:
                sends[0].wait_send()
            if _DO_COMPUTE:
                sendbuf[slot, :, :] = block(j)
            else:
                sendbuf[slot, :, :] = jnp.full(
                    (M_PER, N_PER), 0.5, dtype=jnp.float32
                )
            if _DO_COMM:
                rdma = pltpu.make_async_remote_copy(
                    src_ref=sendbuf.at[slot],
                    dst_ref=out_ref.at[pl.ds(my * M_PER, M_PER), :],
                    send_sem=send_sems.at[t - 1],
                    recv_sem=recv_sems.at[t - 1],
                    device_id=(j,),
                    device_id_type=pl.DeviceIdType.MESH,
                )
                rdma.start()
                sends.append(rdma)

        if _DO_COMPUTE:
            out_ref[pl.ds(my * M_PER, M_PER), :] = block(my)
        else:
            out_ref[pl.ds(my * M_PER, M_PER), :] = sendbuf[0, :, :]

        for rdma in sends[1:]:
            rdma.wait_send()
        if _DO_COMM:
            for t in range(1, N_DEV):
                src = (my - t) % N_DEV
                recv = pltpu.make_async_remote_copy(
                    src_ref=sendbuf.at[0],
                    dst_ref=out_ref.at[pl.ds(src * M_PER, M_PER), :],
                    send_sem=send_sems.at[0],
                    recv_sem=recv_sems.at[t - 1],
                    device_id=(src,),
                    device_id_type=pl.DeviceIdType.MESH,
                )
                recv.wait_recv()

    return pl.pallas_call(
        body,
        out_shape=jax.ShapeDtypeStruct((N_DEV * M_PER, N_PER), jnp.float32),
        in_specs=[
            pl.BlockSpec(memory_space=pltpu.VMEM),
            pl.BlockSpec(memory_space=pltpu.VMEM),
            pl.BlockSpec(memory_space=pltpu.SMEM),
        ],
        out_specs=pl.BlockSpec(memory_space=pltpu.VMEM),
        scratch_shapes=[
            pltpu.VMEM((6, M_PER, N_PER), jnp.float32),
            pltpu.SemaphoreType.DMA((N_DEV - 1,)),
            pltpu.SemaphoreType.DMA((N_DEV - 1,)),
        ],
        compiler_params=pltpu.CompilerParams(
            vmem_limit_bytes=128 * 1024 * 1024,
        ),
    )(x8, w8, scale)


# device time: 100138 ns/iter; 2.3103x vs baseline; 2.3103x over previous
import os

import jax
import jax.numpy as jnp
from jax import lax
from jax.experimental import pallas as pl
from jax.experimental.pallas import tpu as pltpu

N_DEV = 8
M_PER = 512
N_PER = 1024
K = 4096
HCOL = 512

_KVAR = os.environ.get("KVAR", "full")
_DO_COMPUTE = _KVAR != "comm"
_DO_COMM = _KVAR != "compute"


def kernel(x, w_mat, scale_x, scale_w):
    x8 = x.astype(jnp.float8_e5m2)
    scale = (scale_x[0] * scale_w[0]).reshape(1, 1).astype(jnp.float32)

    steps = [(t, h) for t in range(1, N_DEV + 1) for h in range(2)]

    def body(x_ref, w_hbm, s_ref, out_ref, wf32, w8, sendbuf, recvbuf,
             wdma_sems, send_sems, recv_sems):
        my = lax.axis_index("i")
        s = s_ref[0, 0]

        def start_wdma(si):
            t, h = steps[si]
            col = ((my + t) % N_DEV) * N_PER + h * HCOL
            cp = pltpu.make_async_copy(
                w_hbm.at[:, pl.ds(col, HCOL)],
                wf32.at[si % 2],
                wdma_sems.at[si % 2],
            )
            cp.start()
            return cp

        sends = []
        if _DO_COMPUTE:
            dmas = [start_wdma(0)]
            for si, (t, h) in enumerate(steps):
                slot = si % 2
                if si + 1 < len(steps):
                    dmas.append(start_wdma(si + 1))
                dmas[si].wait()
                w8[slot, :, :] = wf32[slot, :, :].astype(jnp.float8_e5m2)
                acc = jnp.dot(
                    x_ref[...], w8[slot, :, :],
                    preferred_element_type=jnp.float32,
                )
                y = acc * s
                z = y * jax.nn.sigmoid(y)
                if t < N_DEV:
                    sendbuf[t - 1, :, h * HCOL:(h + 1) * HCOL] = z.astype(
                        jnp.bfloat16
                    )
                else:
                    out_ref[pl.ds(my * M_PER, M_PER),
                            h * HCOL:(h + 1) * HCOL] = z
                if _DO_COMM and t < N_DEV and h == 1:
                    rdma = pltpu.make_async_remote_copy(
                        src_ref=sendbuf.at[t - 1],
                        dst_ref=recvbuf.at[t - 1],
                        send_sem=send_sems.at[t - 1],
                        recv_sem=recv_sems.at[t - 1],
                        device_id=((my + t) % N_DEV,),
                        device_id_type=pl.DeviceIdType.MESH,
                    )
                    rdma.start()
                    sends.append(rdma)
        else:
            for t in range(1, N_DEV):
                sendbuf[t - 1, :, :] = jnp.full(
                    (M_PER, N_PER), 0.5, dtype=jnp.bfloat16
                )
                rdma = pltpu.make_async_remote_copy(
                    src_ref=sendbuf.at[t - 1],
                    dst_ref=recvbuf.at[t - 1],
                    send_sem=send_sems.at[t - 1],
                    recv_sem=recv_sems.at[t - 1],
                    device_id=((my + t) % N_DEV,),
                    device_id_type=pl.DeviceIdType.MESH,
                )
                rdma.start()
                sends.append(rdma)
            out_ref[pl.ds(my * M_PER, M_PER), :] = jnp.zeros(
                (M_PER, N_PER), dtype=jnp.float32
            )

        if _DO_COMM:
            for t in range(1, N_DEV):
                src = (my - t) % N_DEV
                recv = pltpu.make_async_remote_copy(
                    src_ref=sendbuf.at[t - 1],
                    dst_ref=recvbuf.at[t - 1],
                    send_sem=send_sems.at[t - 1],
                    recv_sem=recv_sems.at[t - 1],
                    device_id=(src,),
                    device_id_type=pl.DeviceIdType.MESH,
                )
                recv.wait_recv()
                out_ref[pl.ds(src * M_PER, M_PER), :] = recvbuf[
                    t - 1, :, :
                ].astype(jnp.float32)
            for rdma in sends:
                rdma.wait_send()

    return pl.pallas_call(
        body,
        out_shape=jax.ShapeDtypeStruct((N_DEV * M_PER, N_PER), jnp.float32),
        in_specs=[
            pl.BlockSpec(memory_space=pltpu.VMEM),
            pl.BlockSpec(memory_space=pl.ANY),
            pl.BlockSpec(memory_space=pltpu.SMEM),
        ],
        out_specs=pl.BlockSpec(memory_space=pltpu.VMEM),
        scratch_shapes=[
            pltpu.VMEM((2, K, HCOL), jnp.float32),
            pltpu.VMEM((2, K, HCOL), jnp.float8_e5m2),
            pltpu.VMEM((N_DEV - 1, M_PER, N_PER), jnp.bfloat16),
            pltpu.VMEM((N_DEV - 1, M_PER, N_PER), jnp.bfloat16),
            pltpu.SemaphoreType.DMA((2,)),
            pltpu.SemaphoreType.DMA((N_DEV - 1,)),
            pltpu.SemaphoreType.DMA((N_DEV - 1,)),
        ],
        compiler_params=pltpu.CompilerParams(
            vmem_limit_bytes=128 * 1024 * 1024,
        ),
    )(x8, w_mat, scale)
